# baseline (speedup 1.0000x reference)
import functools

import jax
import jax.numpy as jnp
from jax import lax
from jax.experimental import pallas as pl
from jax.experimental.pallas import tpu as pltpu
from jax.experimental.pallas import tpu_sc as plsc

N = 10000
D = 128
E = 320000
EL = 200000
NC = 2
NS = 16
L = 16

EC = 80
ET = E // NS
TCH = ET // EC
KB = 10
CPD = E // EC
NP = 10240
SEG = NP // NS
RB = 32

_mesh = plsc.VectorSubcoreMesh(core_axis_name="c", subcore_axis_name="s")


def _agg_body(with_cnt, *refs):
    if with_cnt:
        (srcs, dsts, table, mean_out, inv_out, acc, cnt, sb, db, r0, r1,
         f0, f1, ones, zb, cb, gs0, gs1, ss0, ss1, osem) = refs
    else:
        (srcs, dsts, table, inv_in, mean_out, acc, sb, db, r0, r1,
         f0, f1, zb, cb, gs0, gs1, ss0, ss1) = refs
    d = lax.axis_index("c")
    s = lax.axis_index("s")
    z16 = jnp.zeros((L,), jnp.float32)
    o16 = jnp.ones((L,), jnp.float32)

    def fill_z(r, _):
        for j in range(D // L):
            zb[r, pl.ds(L * j, L)] = z16
        return _
    lax.fori_loop(0, RB, fill_z, None)

    def fill_c(r, _):
        cb[r, pl.ds(0, L)] = z16
        return _
    lax.fori_loop(0, RB, fill_c, None)

    if with_cnt:
        def fill_o(r, _):
            ones[r, pl.ds(0, L)] = o16
            return _
        lax.fori_loop(0, EC, fill_o, None)

    for k in range(SEG // RB):
        pltpu.sync_copy(zb, acc.at[pl.ds(s * SEG + RB * k, RB)])
        if with_cnt:
            pltpu.sync_copy(cb, cnt.at[pl.ds(s * SEG + RB * k, RB)])
    plsc.subcore_barrier()

    rbuf = (r0, r1)
    fbuf = (f0, f1)
    gsem = (gs0, gs1)
    ssem = (ss0, ss1)
    himask = jnp.full((L,), -65536, jnp.int32)

    def widen(rsrc, fdst):
        def conv(r, _):
            for g in range(D // 32):
                v = rsrc[r, pl.ds(16 * g, L)]
                lo = lax.bitcast_convert_type(
                    lax.shift_left(v, 16), jnp.float32)
                hi = lax.bitcast_convert_type(v & himask, jnp.float32)
                fdst[r, pl.ds(32 * g, L)] = lo
                fdst[r, pl.ds(32 * g + L, L)] = hi
            return _
        lax.fori_loop(0, EC, conv, None)

    def block(b, _):
        row = d * CPD + s * TCH + b * KB
        pltpu.sync_copy(srcs.at[pl.ds(row, KB)], sb)
        pltpu.sync_copy(dsts.at[pl.ds(row, KB)], db)
        pg = [None, None]
        psc = [None, None]
        pon = [None, None]
        pg[0] = pltpu.async_copy(table.at[sb.at[0]], r0, gs0)
        for jj in range(KB):
            cur = jj % 2
            nxt = 1 - cur
            if jj + 1 < KB:
                if psc[nxt] is not None:
                    psc[nxt].wait()
                    psc[nxt] = None
                    if with_cnt:
                        pon[nxt].wait()
                pg[nxt] = pltpu.async_copy(
                    table.at[sb.at[jj + 1]], rbuf[nxt], gsem[nxt])
            pg[cur].wait()
            widen(rbuf[cur], fbuf[cur])
            psc[cur] = pltpu.async_copy(
                fbuf[cur], acc.at[db.at[jj]], ssem[cur], add=True)
            if with_cnt:
                pon[cur] = pltpu.async_copy(
                    ones, cnt.at[db.at[jj]], osem, add=True)
        for kk in (0, 1):
            if psc[kk] is not None:
                psc[kk].wait()
                if with_cnt:
                    pon[kk].wait()
        return _
    lax.fori_loop(0, TCH // KB, block, None)
    plsc.subcore_barrier()

    for k in range(SEG // RB):
        rb = s * SEG + RB * k
        pltpu.sync_copy(acc.at[pl.ds(rb, RB)], zb)
        if with_cnt:
            pltpu.sync_copy(cnt.at[pl.ds(rb, RB)], cb)
        else:
            pltpu.sync_copy(inv_in.at[pl.ds(d * NP + rb, RB)], cb)

        def row_mean(r, _):
            cv = cb[r, pl.ds(0, L)]
            if with_cnt:
                inv = 1.0 / jnp.maximum(cv, 1.0)
                cb[r, pl.ds(0, L)] = inv
            else:
                inv = cv
            for j in range(D // L):
                sl = pl.ds(L * j, L)
                zb[r, sl] = zb[r, sl] * inv
            return _
        lax.fori_loop(0, RB, row_mean, None)
        pltpu.sync_copy(zb, mean_out.at[pl.ds(d * NP + rb, RB)])
        if with_cnt:
            pltpu.sync_copy(cb, inv_out.at[pl.ds(d * NP + rb, RB)])


_agg0 = pl.kernel(
    functools.partial(_agg_body, True),
    out_type=[jax.ShapeDtypeStruct((2 * NP, D), jnp.float32),
              jax.ShapeDtypeStruct((2 * NP, L), jnp.float32)],
    mesh=_mesh,
    scratch_types=[
        pltpu.VMEM_SHARED((NP, D), jnp.float32),
        pltpu.VMEM_SHARED((NP, L), jnp.float32),
        pltpu.VMEM((KB, EC), jnp.int32),
        pltpu.VMEM((KB, EC), jnp.int32),
        pltpu.VMEM((EC, D // 2), jnp.int32),
        pltpu.VMEM((EC, D // 2), jnp.int32),
        pltpu.VMEM((EC, D), jnp.float32),
        pltpu.VMEM((EC, D), jnp.float32),
        pltpu.VMEM((EC, L), jnp.float32),
        pltpu.VMEM((RB, D), jnp.float32),
        pltpu.VMEM((RB, L), jnp.float32),
        pltpu.SemaphoreType.DMA,
        pltpu.SemaphoreType.DMA,
        pltpu.SemaphoreType.DMA,
        pltpu.SemaphoreType.DMA,
        pltpu.SemaphoreType.DMA,
    ],
    compiler_params=pltpu.CompilerParams(use_tc_tiling_on_sc=False),
)

_agg1 = pl.kernel(
    functools.partial(_agg_body, False),
    out_type=jax.ShapeDtypeStruct((2 * NP, D), jnp.float32),
    mesh=_mesh,
    scratch_types=[
        pltpu.VMEM_SHARED((NP, D), jnp.float32),
        pltpu.VMEM((KB, EC), jnp.int32),
        pltpu.VMEM((KB, EC), jnp.int32),
        pltpu.VMEM((EC, D // 2), jnp.int32),
        pltpu.VMEM((EC, D // 2), jnp.int32),
        pltpu.VMEM((EC, D), jnp.float32),
        pltpu.VMEM((EC, D), jnp.float32),
        pltpu.VMEM((RB, D), jnp.float32),
        pltpu.VMEM((RB, L), jnp.float32),
        pltpu.SemaphoreType.DMA,
        pltpu.SemaphoreType.DMA,
        pltpu.SemaphoreType.DMA,
        pltpu.SemaphoreType.DMA,
    ],
    compiler_params=pltpu.CompilerParams(use_tc_tiling_on_sc=False),
)


DC = 80
DCH = EL // DC
DW = NC * NS
DCT = DCH // DW
CB = 13
DNB = DCT // CB


def _dec_chunk(sidx_r, didx_r, srows, drows, obuf):
    lane = lax.iota(jnp.int32, L)

    def group(g, _):
        def row16(r16, res):
            r = g * L + r16
            acc = jnp.zeros((L,), jnp.float32)
            for j in range(D // L):
                sl2 = pl.ds(L * j, L)
                acc = acc + srows[r, sl2] * drows[r, sl2]
            dot = jnp.sum(acc)
            return jnp.where(lane == r16, dot, res)
        res = lax.fori_loop(0, L, row16, jnp.zeros((L,), jnp.float32))
        obuf[pl.ds(L * g, L)] = res
        return _
    lax.fori_loop(0, DC // L, group, None)


def _dec_body(z, ell, out, sb, db, s0, s1, d0, d1, obuf,
              sem_s0, sem_s1, sem_d0, sem_d1):
    c = lax.axis_index("c")
    s = lax.axis_index("s")
    w = s * NC + c
    sbuf = (s0, s1)
    dbuf = (d0, d1)
    ssem = (sem_s0, sem_s1)
    dsem = (sem_d0, sem_d1)

    def block(b, _):
        row0 = w * DCT + b * CB
        pltpu.sync_copy(ell.at[pl.ds(row0, CB)], sb)
        pltpu.sync_copy(ell.at[pl.ds(DCH + row0, CB)], db)
        ps = [None, None]
        pd = [None, None]
        ps[0] = pltpu.async_copy(z.at[sb.at[0]], s0, sem_s0)
        pd[0] = pltpu.async_copy(z.at[db.at[0]], d0, sem_d0)
        for jj in range(CB):
            cur = jj % 2
            if jj + 1 < CB:
                nxt = 1 - cur
                ps[nxt] = pltpu.async_copy(
                    z.at[sb.at[jj + 1]], sbuf[nxt], ssem[nxt])
                pd[nxt] = pltpu.async_copy(
                    z.at[db.at[jj + 1]], dbuf[nxt], dsem[nxt])
            ps[cur].wait()
            pd[cur].wait()
            _dec_chunk(None, None, sbuf[cur], dbuf[cur], obuf)
            pltpu.sync_copy(obuf, out.at[pl.ds((row0 + jj) * DC, DC)])
        return _
    lax.fori_loop(0, DNB, block, None)

    crow = jnp.where(w < DCH - DW * DCT, DW * DCT + w, w * DCT + DCT - 1)
    pltpu.sync_copy(ell.at[crow], sb.at[0])
    pltpu.sync_copy(ell.at[DCH + crow], db.at[0])
    pltpu.async_copy(z.at[sb.at[0]], s0, sem_s0).wait()
    pltpu.async_copy(z.at[db.at[0]], d0, sem_d0).wait()
    _dec_chunk(None, None, s0, d0, obuf)
    pltpu.sync_copy(obuf, out.at[pl.ds(crow * DC, DC)])


_dec = pl.kernel(
    _dec_body,
    out_type=jax.ShapeDtypeStruct((EL,), jnp.float32),
    mesh=_mesh,
    scratch_types=[
        pltpu.VMEM((CB, DC), jnp.int32),
        pltpu.VMEM((CB, DC), jnp.int32),
        pltpu.VMEM((DC, D), jnp.float32),
        pltpu.VMEM((DC, D), jnp.float32),
        pltpu.VMEM((DC, D), jnp.float32),
        pltpu.VMEM((DC, D), jnp.float32),
        pltpu.VMEM((DC,), jnp.float32),
        pltpu.SemaphoreType.DMA,
        pltpu.SemaphoreType.DMA,
        pltpu.SemaphoreType.DMA,
        pltpu.SemaphoreType.DMA,
    ],
    compiler_params=pltpu.CompilerParams(
        use_tc_tiling_on_sc=False, needs_layout_passes=False),
)


BN = 1000
NBP = N // BN


def _dense(mean3, x_flat, wm2, wr2, b2, relu):
    def body(m_ref, x_ref, wm_ref, wr_ref, b_ref, o_ref):
        acc = jnp.dot(m_ref[0], wm_ref[0], preferred_element_type=jnp.float32)
        acc = acc + jnp.dot(x_ref[...], wr_ref[0],
                            preferred_element_type=jnp.float32)
        acc = acc + b_ref[0]
        if relu:
            acc = jnp.maximum(acc, 0.0)
        o_ref[...] = acc

    return pl.pallas_call(
        body,
        grid=(2, NBP),
        in_specs=[
            pl.BlockSpec((1, BN, D), lambda h, i: (1 - h, i, 0)),
            pl.BlockSpec((BN, D), lambda h, i: (h * NBP + i, 0)),
            pl.BlockSpec((1, D, D), lambda h, i: (h, 0, 0)),
            pl.BlockSpec((1, D, D), lambda h, i: (h, 0, 0)),
            pl.BlockSpec((1, 1, D), lambda h, i: (h, 0, 0)),
        ],
        out_specs=pl.BlockSpec((BN, D), lambda h, i: (h * NBP + i, 0)),
        out_shape=jax.ShapeDtypeStruct((2 * N, D), jnp.float32),
    )(mean3, x_flat, wm2, wr2, b2)


def kernel(emb_user, emb_item,
           W_msg_ui_0, W_root_ui_0, b_ui_0,
           W_msg_iu_0, W_root_iu_0, b_iu_0,
           W_msg_ui_1, W_root_ui_1, b_ui_1,
           W_msg_iu_1, W_root_iu_1, b_iu_1,
           edge_index_ui, edge_index_iu, edge_label_index):
    table0 = jnp.concatenate([emb_user, emb_item], axis=0)
    srcs = jnp.concatenate([
        edge_index_ui[0], edge_index_iu[0] + N]).astype(jnp.int32)
    srcs = srcs.reshape(2 * CPD, EC)
    dsts = jnp.concatenate([
        edge_index_ui[1], edge_index_iu[1]]).astype(jnp.int32)
    dsts = dsts.reshape(2 * CPD, EC)
    ell2d = jnp.concatenate(
        [edge_label_index[0],
         edge_label_index[1] + N]).astype(jnp.int32).reshape(2 * DCH, DC)

    perm = []
    for g in range(D // 32):
        perm += [32 * g + 2 * k for k in range(16)]
        perm += [32 * g + 2 * k + 1 for k in range(16)]
    perm = jnp.array(perm, dtype=jnp.int32)
    wm0 = jnp.stack([W_msg_iu_0, W_msg_ui_0])[:, perm, :]
    wr0 = jnp.stack([W_root_iu_0, W_root_ui_0])
    bb0 = jnp.stack([b_iu_0, b_ui_0]).reshape(2, 1, D)
    wm1 = jnp.stack([W_msg_iu_1, W_msg_ui_1])[:, perm, :]
    wr1 = jnp.stack([W_root_iu_1, W_root_ui_1])
    bb1 = jnp.stack([b_iu_1, b_ui_1]).reshape(2, 1, D)

    def packbf(t):
        b = t.astype(jnp.bfloat16).reshape(2 * N, D // 2, 2)
        return jax.lax.bitcast_convert_type(b, jnp.int32)

    mean0, inv = _agg0(srcs, dsts, packbf(table0))
    mean0 = mean0.reshape(2, NP, D)
    table1 = _dense(mean0, table0, wm0, wr0, bb0, True)
    mean1 = _agg1(srcs, dsts, packbf(table1), inv).reshape(2, NP, D)
    z_flat = _dense(mean1, table1, wm1, wr1, bb1, False)
    return _dec(z_flat, ell2d)


# speedup vs baseline: 1.9231x; 1.9231x over previous
import functools

import jax
import jax.numpy as jnp
from jax import lax
from jax.experimental import pallas as pl
from jax.experimental.pallas import tpu as pltpu
from jax.experimental.pallas import tpu_sc as plsc

N = 10000
D = 128
E = 320000
EL = 200000
NC = 2
NS = 16
L = 16

EC = 80
ET = E // NS
TCH = ET // EC
KB = 10
CPD = E // EC
NP = 10240
SEG = NP // NS
RB = 64

_mesh = plsc.VectorSubcoreMesh(core_axis_name="c", subcore_axis_name="s")


def _agg_body(with_cnt, *refs):
    if with_cnt:
        (srcs, dsts, table, mean_out, inv_out, acc, cnt, sb, db, r0, r1,
         ones, zb, cb, gs0, gs1, ss0, ss1, osem) = refs
    else:
        (srcs, dsts, table, inv_in, mean_out, acc, sb, db, r0, r1,
         zb, cb, gs0, gs1, ss0, ss1) = refs
    d = lax.axis_index("c")
    s = lax.axis_index("s")
    z16 = jnp.zeros((L,), jnp.float32)
    o16 = jnp.ones((L,), jnp.float32)

    def fill_z(r, _):
        for j in range(D // L):
            zb[r, pl.ds(L * j, L)] = z16
        return _
    lax.fori_loop(0, RB, fill_z, None)

    def fill_c(r, _):
        cb[r, pl.ds(0, L)] = z16
        return _
    lax.fori_loop(0, RB, fill_c, None)

    if with_cnt:
        def fill_o(r, _):
            ones[r, pl.ds(0, L)] = o16
            return _
        lax.fori_loop(0, EC, fill_o, None)

    for k in range(SEG // RB):
        pltpu.sync_copy(zb, acc.at[pl.ds(s * SEG + RB * k, RB)])
        if with_cnt:
            pltpu.sync_copy(cb, cnt.at[pl.ds(s * SEG + RB * k, RB)])
    plsc.subcore_barrier()

    rbuf = (r0, r1)
    gsem = (gs0, gs1)
    ssem = (ss0, ss1)

    def block(b, _):
        row = d * CPD + s * TCH + b * KB
        pltpu.sync_copy(srcs.at[pl.ds(row, KB)], sb)
        pltpu.sync_copy(dsts.at[pl.ds(row, KB)], db)
        pg = [None, None]
        psc = [None, None]
        pon = [None, None]
        pg[0] = pltpu.async_copy(table.at[sb.at[0]], r0, gs0)
        for jj in range(KB):
            cur = jj % 2
            nxt = 1 - cur
            if jj + 1 < KB:
                if psc[nxt] is not None:
                    psc[nxt].wait()
                    psc[nxt] = None
                    if with_cnt:
                        pon[nxt].wait()
                pg[nxt] = pltpu.async_copy(
                    table.at[sb.at[jj + 1]], rbuf[nxt], gsem[nxt])
            pg[cur].wait()
            psc[cur] = pltpu.async_copy(
                rbuf[cur], acc.at[db.at[jj]], ssem[cur], add=True)
            if with_cnt:
                pon[cur] = pltpu.async_copy(
                    ones, cnt.at[db.at[jj]], osem, add=True)
        for kk in (0, 1):
            if psc[kk] is not None:
                psc[kk].wait()
                if with_cnt:
                    pon[kk].wait()
        return _
    lax.fori_loop(0, TCH // KB, block, None)
    plsc.subcore_barrier()

    for k in range(SEG // RB):
        rb = s * SEG + RB * k
        pltpu.sync_copy(acc.at[pl.ds(rb, RB)], zb)
        if with_cnt:
            pltpu.sync_copy(cnt.at[pl.ds(rb, RB)], cb)
        else:
            pltpu.sync_copy(inv_in.at[pl.ds(d * NP + rb, RB)], cb)

        def row_mean(r, _):
            cv = cb[r, pl.ds(0, L)]
            if with_cnt:
                inv = 1.0 / jnp.maximum(cv, 1.0)
                cb[r, pl.ds(0, L)] = inv
            else:
                inv = cv
            for j in range(D // L):
                sl = pl.ds(L * j, L)
                zb[r, sl] = zb[r, sl] * inv
            return _
        lax.fori_loop(0, RB, row_mean, None)
        pltpu.sync_copy(zb, mean_out.at[pl.ds(d * NP + rb, RB)])
        if with_cnt:
            pltpu.sync_copy(cb, inv_out.at[pl.ds(d * NP + rb, RB)])


_agg0 = pl.kernel(
    functools.partial(_agg_body, True),
    out_type=[jax.ShapeDtypeStruct((2 * NP, D), jnp.float32),
              jax.ShapeDtypeStruct((2 * NP, L), jnp.float32)],
    mesh=_mesh,
    scratch_types=[
        pltpu.VMEM_SHARED((NP, D), jnp.float32),
        pltpu.VMEM_SHARED((NP, L), jnp.float32),
        pltpu.VMEM((KB, EC), jnp.int32),
        pltpu.VMEM((KB, EC), jnp.int32),
        pltpu.VMEM((EC, D), jnp.float32),
        pltpu.VMEM((EC, D), jnp.float32),
        pltpu.VMEM((EC, L), jnp.float32),
        pltpu.VMEM((RB, D), jnp.float32),
        pltpu.VMEM((RB, L), jnp.float32),
        pltpu.SemaphoreType.DMA,
        pltpu.SemaphoreType.DMA,
        pltpu.SemaphoreType.DMA,
        pltpu.SemaphoreType.DMA,
        pltpu.SemaphoreType.DMA,
    ],
    compiler_params=pltpu.CompilerParams(use_tc_tiling_on_sc=False),
)

_agg1 = pl.kernel(
    functools.partial(_agg_body, False),
    out_type=jax.ShapeDtypeStruct((2 * NP, D), jnp.float32),
    mesh=_mesh,
    scratch_types=[
        pltpu.VMEM_SHARED((NP, D), jnp.float32),
        pltpu.VMEM((KB, EC), jnp.int32),
        pltpu.VMEM((KB, EC), jnp.int32),
        pltpu.VMEM((EC, D), jnp.float32),
        pltpu.VMEM((EC, D), jnp.float32),
        pltpu.VMEM((RB, D), jnp.float32),
        pltpu.VMEM((RB, L), jnp.float32),
        pltpu.SemaphoreType.DMA,
        pltpu.SemaphoreType.DMA,
        pltpu.SemaphoreType.DMA,
        pltpu.SemaphoreType.DMA,
    ],
    compiler_params=pltpu.CompilerParams(use_tc_tiling_on_sc=False),
)


DC = 80
DCH = EL // DC
DW = NC * NS
DCT = DCH // DW
CB = 13
DNB = DCT // CB


def _dec_chunk(sidx_r, didx_r, srows, drows, obuf):
    lane = lax.iota(jnp.int32, L)

    def group(g, _):
        def row16(r16, res):
            r = g * L + r16
            acc = jnp.zeros((L,), jnp.float32)
            for j in range(D // L):
                sl2 = pl.ds(L * j, L)
                acc = acc + srows[r, sl2] * drows[r, sl2]
            dot = jnp.sum(acc)
            return jnp.where(lane == r16, dot, res)
        res = lax.fori_loop(0, L, row16, jnp.zeros((L,), jnp.float32))
        obuf[pl.ds(L * g, L)] = res
        return _
    lax.fori_loop(0, DC // L, group, None)


def _dec_body(z, ell, out, sb, db, s0, s1, d0, d1, obuf,
              sem_s0, sem_s1, sem_d0, sem_d1):
    c = lax.axis_index("c")
    s = lax.axis_index("s")
    w = s * NC + c
    sbuf = (s0, s1)
    dbuf = (d0, d1)
    ssem = (sem_s0, sem_s1)
    dsem = (sem_d0, sem_d1)

    def block(b, _):
        row0 = w * DCT + b * CB
        pltpu.sync_copy(ell.at[pl.ds(row0, CB)], sb)
        pltpu.sync_copy(ell.at[pl.ds(DCH + row0, CB)], db)
        ps = [None, None]
        pd = [None, None]
        ps[0] = pltpu.async_copy(z.at[sb.at[0]], s0, sem_s0)
        pd[0] = pltpu.async_copy(z.at[db.at[0]], d0, sem_d0)
        for jj in range(CB):
            cur = jj % 2
            if jj + 1 < CB:
                nxt = 1 - cur
                ps[nxt] = pltpu.async_copy(
                    z.at[sb.at[jj + 1]], sbuf[nxt], ssem[nxt])
                pd[nxt] = pltpu.async_copy(
                    z.at[db.at[jj + 1]], dbuf[nxt], dsem[nxt])
            ps[cur].wait()
            pd[cur].wait()
            _dec_chunk(None, None, sbuf[cur], dbuf[cur], obuf)
            pltpu.sync_copy(obuf, out.at[pl.ds((row0 + jj) * DC, DC)])
        return _
    lax.fori_loop(0, DNB, block, None)

    crow = jnp.where(w < DCH - DW * DCT, DW * DCT + w, w * DCT + DCT - 1)
    pltpu.sync_copy(ell.at[crow], sb.at[0])
    pltpu.sync_copy(ell.at[DCH + crow], db.at[0])
    pltpu.async_copy(z.at[sb.at[0]], s0, sem_s0).wait()
    pltpu.async_copy(z.at[db.at[0]], d0, sem_d0).wait()
    _dec_chunk(None, None, s0, d0, obuf)
    pltpu.sync_copy(obuf, out.at[pl.ds(crow * DC, DC)])


_dec = pl.kernel(
    _dec_body,
    out_type=jax.ShapeDtypeStruct((EL,), jnp.float32),
    mesh=_mesh,
    scratch_types=[
        pltpu.VMEM((CB, DC), jnp.int32),
        pltpu.VMEM((CB, DC), jnp.int32),
        pltpu.VMEM((DC, D), jnp.float32),
        pltpu.VMEM((DC, D), jnp.float32),
        pltpu.VMEM((DC, D), jnp.float32),
        pltpu.VMEM((DC, D), jnp.float32),
        pltpu.VMEM((DC,), jnp.float32),
        pltpu.SemaphoreType.DMA,
        pltpu.SemaphoreType.DMA,
        pltpu.SemaphoreType.DMA,
        pltpu.SemaphoreType.DMA,
    ],
    compiler_params=pltpu.CompilerParams(
        use_tc_tiling_on_sc=False, needs_layout_passes=False),
)


BN = 1000
NBP = N // BN


def _dense(mean3, x_flat, wm2, wr2, b2, relu):
    def body(m_ref, x_ref, wm_ref, wr_ref, b_ref, o_ref):
        acc = jnp.dot(m_ref[0], wm_ref[0], preferred_element_type=jnp.float32)
        acc = acc + jnp.dot(x_ref[...], wr_ref[0],
                            preferred_element_type=jnp.float32)
        acc = acc + b_ref[0]
        if relu:
            acc = jnp.maximum(acc, 0.0)
        o_ref[...] = acc

    return pl.pallas_call(
        body,
        grid=(2, NBP),
        in_specs=[
            pl.BlockSpec((1, BN, D), lambda h, i: (1 - h, i, 0)),
            pl.BlockSpec((BN, D), lambda h, i: (h * NBP + i, 0)),
            pl.BlockSpec((1, D, D), lambda h, i: (h, 0, 0)),
            pl.BlockSpec((1, D, D), lambda h, i: (h, 0, 0)),
            pl.BlockSpec((1, 1, D), lambda h, i: (h, 0, 0)),
        ],
        out_specs=pl.BlockSpec((BN, D), lambda h, i: (h * NBP + i, 0)),
        out_shape=jax.ShapeDtypeStruct((2 * N, D), jnp.float32),
    )(mean3, x_flat, wm2, wr2, b2)


def kernel(emb_user, emb_item,
           W_msg_ui_0, W_root_ui_0, b_ui_0,
           W_msg_iu_0, W_root_iu_0, b_iu_0,
           W_msg_ui_1, W_root_ui_1, b_ui_1,
           W_msg_iu_1, W_root_iu_1, b_iu_1,
           edge_index_ui, edge_index_iu, edge_label_index):
    table0 = jnp.concatenate([emb_user, emb_item], axis=0)
    srcs = jnp.concatenate([
        edge_index_ui[0], edge_index_iu[0] + N]).astype(jnp.int32)
    srcs = srcs.reshape(2 * CPD, EC)
    dsts = jnp.concatenate([
        edge_index_ui[1], edge_index_iu[1]]).astype(jnp.int32)
    dsts = dsts.reshape(2 * CPD, EC)
    ell2d = jnp.concatenate(
        [edge_label_index[0],
         edge_label_index[1] + N]).astype(jnp.int32).reshape(2 * DCH, DC)

    wm0 = jnp.stack([W_msg_iu_0, W_msg_ui_0])
    wr0 = jnp.stack([W_root_iu_0, W_root_ui_0])
    bb0 = jnp.stack([b_iu_0, b_ui_0]).reshape(2, 1, D)
    wm1 = jnp.stack([W_msg_iu_1, W_msg_ui_1])
    wr1 = jnp.stack([W_root_iu_1, W_root_ui_1])
    bb1 = jnp.stack([b_iu_1, b_ui_1]).reshape(2, 1, D)

    mean0, inv = _agg0(srcs, dsts, table0)
    mean0 = mean0.reshape(2, NP, D)
    table1 = _dense(mean0, table0, wm0, wr0, bb0, True)
    mean1 = _agg1(srcs, dsts, table1, inv).reshape(2, NP, D)
    z_flat = _dense(mean1, table1, wm1, wr1, bb1, False)
    return _dec(z_flat, ell2d)


# speedup vs baseline: 1.9737x; 1.0263x over previous
import functools

import jax
import jax.numpy as jnp
from jax import lax
from jax.experimental import pallas as pl
from jax.experimental.pallas import tpu as pltpu
from jax.experimental.pallas import tpu_sc as plsc

N = 10000
D = 128
E = 320000
EL = 200000
NC = 2
NS = 16
L = 16

EC = 100
ET = E // NS
TCH = ET // EC
KB = 10
CPD = E // EC
NP = 10240
SEG = NP // NS
RB = 32

_mesh = plsc.VectorSubcoreMesh(core_axis_name="c", subcore_axis_name="s")


def _agg_body(with_cnt, *refs):
    if with_cnt:
        (srcs, dsts, table, mean_out, inv_out, acc, cnt, sb, db, r0, r1,
         ones, zb, cb, gs0, gs1, ss0, ss1, osem) = refs
    else:
        (srcs, dsts, table, inv_in, mean_out, acc, sb, db, r0, r1,
         zb, cb, gs0, gs1, ss0, ss1) = refs
    d = lax.axis_index("c")
    s = lax.axis_index("s")
    z16 = jnp.zeros((L,), jnp.float32)
    o16 = jnp.ones((L,), jnp.float32)

    def fill_z(r, _):
        for j in range(D // L):
            zb[r, pl.ds(L * j, L)] = z16
        return _
    lax.fori_loop(0, RB, fill_z, None)

    def fill_c(r, _):
        cb[r, pl.ds(0, L)] = z16
        return _
    lax.fori_loop(0, RB, fill_c, None)

    if with_cnt:
        def fill_o(r, _):
            ones[r, pl.ds(0, L)] = o16
            return _
        lax.fori_loop(0, EC, fill_o, None)

    for k in range(SEG // RB):
        pltpu.sync_copy(zb, acc.at[pl.ds(s * SEG + RB * k, RB)])
        if with_cnt:
            pltpu.sync_copy(cb, cnt.at[pl.ds(s * SEG + RB * k, RB)])
    plsc.subcore_barrier()

    rbuf = (r0, r1)
    gsem = (gs0, gs1)
    ssem = (ss0, ss1)

    def block(b, _):
        row = d * CPD + s * TCH + b * KB
        pltpu.sync_copy(srcs.at[pl.ds(row, KB)], sb)
        pltpu.sync_copy(dsts.at[pl.ds(row, KB)], db)
        pg = [None, None]
        psc = [None, None]
        pon = [None, None]
        pg[0] = pltpu.async_copy(table.at[sb.at[0]], r0, gs0)
        for jj in range(KB):
            cur = jj % 2
            nxt = 1 - cur
            if jj + 1 < KB:
                if psc[nxt] is not None:
                    psc[nxt].wait()
                    psc[nxt] = None
                    if with_cnt:
                        pon[nxt].wait()
                pg[nxt] = pltpu.async_copy(
                    table.at[sb.at[jj + 1]], rbuf[nxt], gsem[nxt])
            pg[cur].wait()
            psc[cur] = pltpu.async_copy(
                rbuf[cur], acc.at[db.at[jj]], ssem[cur], add=True)
            if with_cnt:
                pon[cur] = pltpu.async_copy(
                    ones, cnt.at[db.at[jj]], osem, add=True)
        for kk in (0, 1):
            if psc[kk] is not None:
                psc[kk].wait()
                if with_cnt:
                    pon[kk].wait()
        return _
    lax.fori_loop(0, TCH // KB, block, None)
    plsc.subcore_barrier()

    for k in range(SEG // RB):
        rb = s * SEG + RB * k
        pltpu.sync_copy(acc.at[pl.ds(rb, RB)], zb)
        if with_cnt:
            pltpu.sync_copy(cnt.at[pl.ds(rb, RB)], cb)
        else:
            pltpu.sync_copy(inv_in.at[pl.ds(d * NP + rb, RB)], cb)

        def row_mean(r, _):
            cv = cb[r, pl.ds(0, L)]
            if with_cnt:
                inv = 1.0 / jnp.maximum(cv, 1.0)
                cb[r, pl.ds(0, L)] = inv
            else:
                inv = cv
            for j in range(D // L):
                sl = pl.ds(L * j, L)
                zb[r, sl] = zb[r, sl] * inv
            return _
        lax.fori_loop(0, RB, row_mean, None)
        pltpu.sync_copy(zb, mean_out.at[pl.ds(d * NP + rb, RB)])
        if with_cnt:
            pltpu.sync_copy(cb, inv_out.at[pl.ds(d * NP + rb, RB)])


_agg0 = pl.kernel(
    functools.partial(_agg_body, True),
    out_type=[jax.ShapeDtypeStruct((2 * NP, D), jnp.float32),
              jax.ShapeDtypeStruct((2 * NP, L), jnp.float32)],
    mesh=_mesh,
    scratch_types=[
        pltpu.VMEM_SHARED((NP, D), jnp.float32),
        pltpu.VMEM_SHARED((NP, L), jnp.float32),
        pltpu.VMEM((KB, EC), jnp.int32),
        pltpu.VMEM((KB, EC), jnp.int32),
        pltpu.VMEM((EC, D), jnp.float32),
        pltpu.VMEM((EC, D), jnp.float32),
        pltpu.VMEM((EC, L), jnp.float32),
        pltpu.VMEM((RB, D), jnp.float32),
        pltpu.VMEM((RB, L), jnp.float32),
        pltpu.SemaphoreType.DMA,
        pltpu.SemaphoreType.DMA,
        pltpu.SemaphoreType.DMA,
        pltpu.SemaphoreType.DMA,
        pltpu.SemaphoreType.DMA,
    ],
    compiler_params=pltpu.CompilerParams(use_tc_tiling_on_sc=False),
)

_agg1 = pl.kernel(
    functools.partial(_agg_body, False),
    out_type=jax.ShapeDtypeStruct((2 * NP, D), jnp.float32),
    mesh=_mesh,
    scratch_types=[
        pltpu.VMEM_SHARED((NP, D), jnp.float32),
        pltpu.VMEM((KB, EC), jnp.int32),
        pltpu.VMEM((KB, EC), jnp.int32),
        pltpu.VMEM((EC, D), jnp.float32),
        pltpu.VMEM((EC, D), jnp.float32),
        pltpu.VMEM((RB, D), jnp.float32),
        pltpu.VMEM((RB, L), jnp.float32),
        pltpu.SemaphoreType.DMA,
        pltpu.SemaphoreType.DMA,
        pltpu.SemaphoreType.DMA,
        pltpu.SemaphoreType.DMA,
    ],
    compiler_params=pltpu.CompilerParams(use_tc_tiling_on_sc=False),
)


DC = 80
DCH = EL // DC
DW = NC * NS
DCT = DCH // DW
CB = 13
DNB = DCT // CB


def _dec_chunk(sidx_r, didx_r, srows, drows, obuf):
    lane = lax.iota(jnp.int32, L)

    def group(g, _):
        def row16(r16, res):
            r = g * L + r16
            acc = jnp.zeros((L,), jnp.float32)
            for j in range(D // L):
                sl2 = pl.ds(L * j, L)
                acc = acc + srows[r, sl2] * drows[r, sl2]
            dot = jnp.sum(acc)
            return jnp.where(lane == r16, dot, res)
        res = lax.fori_loop(0, L, row16, jnp.zeros((L,), jnp.float32))
        obuf[pl.ds(L * g, L)] = res
        return _
    lax.fori_loop(0, DC // L, group, None)


def _dec_body(z, ell, out, sb, db, s0, s1, d0, d1, obuf,
              sem_s0, sem_s1, sem_d0, sem_d1):
    c = lax.axis_index("c")
    s = lax.axis_index("s")
    w = s * NC + c
    sbuf = (s0, s1)
    dbuf = (d0, d1)
    ssem = (sem_s0, sem_s1)
    dsem = (sem_d0, sem_d1)

    def block(b, _):
        row0 = w * DCT + b * CB
        pltpu.sync_copy(ell.at[pl.ds(row0, CB)], sb)
        pltpu.sync_copy(ell.at[pl.ds(DCH + row0, CB)], db)
        ps = [None, None]
        pd = [None, None]
        ps[0] = pltpu.async_copy(z.at[sb.at[0]], s0, sem_s0)
        pd[0] = pltpu.async_copy(z.at[db.at[0]], d0, sem_d0)
        for jj in range(CB):
            cur = jj % 2
            if jj + 1 < CB:
                nxt = 1 - cur
                ps[nxt] = pltpu.async_copy(
                    z.at[sb.at[jj + 1]], sbuf[nxt], ssem[nxt])
                pd[nxt] = pltpu.async_copy(
                    z.at[db.at[jj + 1]], dbuf[nxt], dsem[nxt])
            ps[cur].wait()
            pd[cur].wait()
            _dec_chunk(None, None, sbuf[cur], dbuf[cur], obuf)
            pltpu.sync_copy(obuf, out.at[pl.ds((row0 + jj) * DC, DC)])
        return _
    lax.fori_loop(0, DNB, block, None)

    crow = jnp.where(w < DCH - DW * DCT, DW * DCT + w, w * DCT + DCT - 1)
    pltpu.sync_copy(ell.at[crow], sb.at[0])
    pltpu.sync_copy(ell.at[DCH + crow], db.at[0])
    pltpu.async_copy(z.at[sb.at[0]], s0, sem_s0).wait()
    pltpu.async_copy(z.at[db.at[0]], d0, sem_d0).wait()
    _dec_chunk(None, None, s0, d0, obuf)
    pltpu.sync_copy(obuf, out.at[pl.ds(crow * DC, DC)])


_dec = pl.kernel(
    _dec_body,
    out_type=jax.ShapeDtypeStruct((EL,), jnp.float32),
    mesh=_mesh,
    scratch_types=[
        pltpu.VMEM((CB, DC), jnp.int32),
        pltpu.VMEM((CB, DC), jnp.int32),
        pltpu.VMEM((DC, D), jnp.float32),
        pltpu.VMEM((DC, D), jnp.float32),
        pltpu.VMEM((DC, D), jnp.float32),
        pltpu.VMEM((DC, D), jnp.float32),
        pltpu.VMEM((DC,), jnp.float32),
        pltpu.SemaphoreType.DMA,
        pltpu.SemaphoreType.DMA,
        pltpu.SemaphoreType.DMA,
        pltpu.SemaphoreType.DMA,
    ],
    compiler_params=pltpu.CompilerParams(
        use_tc_tiling_on_sc=False, needs_layout_passes=False),
)


BN = 1000
NBP = N // BN


def _dense(mean3, x_flat, wm2, wr2, b2, relu):
    def body(m_ref, x_ref, wm_ref, wr_ref, b_ref, o_ref):
        acc = jnp.dot(m_ref[0], wm_ref[0], preferred_element_type=jnp.float32)
        acc = acc + jnp.dot(x_ref[...], wr_ref[0],
                            preferred_element_type=jnp.float32)
        acc = acc + b_ref[0]
        if relu:
            acc = jnp.maximum(acc, 0.0)
        o_ref[...] = acc

    return pl.pallas_call(
        body,
        grid=(2, NBP),
        in_specs=[
            pl.BlockSpec((1, BN, D), lambda h, i: (1 - h, i, 0)),
            pl.BlockSpec((BN, D), lambda h, i: (h * NBP + i, 0)),
            pl.BlockSpec((1, D, D), lambda h, i: (h, 0, 0)),
            pl.BlockSpec((1, D, D), lambda h, i: (h, 0, 0)),
            pl.BlockSpec((1, 1, D), lambda h, i: (h, 0, 0)),
        ],
        out_specs=pl.BlockSpec((BN, D), lambda h, i: (h * NBP + i, 0)),
        out_shape=jax.ShapeDtypeStruct((2 * N, D), jnp.float32),
    )(mean3, x_flat, wm2, wr2, b2)


def kernel(emb_user, emb_item,
           W_msg_ui_0, W_root_ui_0, b_ui_0,
           W_msg_iu_0, W_root_iu_0, b_iu_0,
           W_msg_ui_1, W_root_ui_1, b_ui_1,
           W_msg_iu_1, W_root_iu_1, b_iu_1,
           edge_index_ui, edge_index_iu, edge_label_index):
    table0 = jnp.concatenate([emb_user, emb_item], axis=0)
    srcs = jnp.concatenate([
        edge_index_ui[0], edge_index_iu[0] + N]).astype(jnp.int32)
    srcs = srcs.reshape(2 * CPD, EC)
    dsts = jnp.concatenate([
        edge_index_ui[1], edge_index_iu[1]]).astype(jnp.int32)
    dsts = dsts.reshape(2 * CPD, EC)
    ell2d = jnp.concatenate(
        [edge_label_index[0],
         edge_label_index[1] + N]).astype(jnp.int32).reshape(2 * DCH, DC)

    wm0 = jnp.stack([W_msg_iu_0, W_msg_ui_0])
    wr0 = jnp.stack([W_root_iu_0, W_root_ui_0])
    bb0 = jnp.stack([b_iu_0, b_ui_0]).reshape(2, 1, D)
    wm1 = jnp.stack([W_msg_iu_1, W_msg_ui_1])
    wr1 = jnp.stack([W_root_iu_1, W_root_ui_1])
    bb1 = jnp.stack([b_iu_1, b_ui_1]).reshape(2, 1, D)

    mean0, inv = _agg0(srcs, dsts, table0)
    mean0 = mean0.reshape(2, NP, D)
    table1 = _dense(mean0, table0, wm0, wr0, bb0, True)
    mean1 = _agg1(srcs, dsts, table1, inv).reshape(2, NP, D)
    z_flat = _dense(mean1, table1, wm1, wr1, bb1, False)
    return _dec(z_flat, ell2d)


# speedup vs baseline: 2.0849x; 1.0563x over previous
import functools

import jax
import jax.numpy as jnp
from jax import lax
from jax.experimental import pallas as pl
from jax.experimental.pallas import tpu as pltpu
from jax.experimental.pallas import tpu_sc as plsc

N = 10000
D = 128
E = 320000
EL = 200000
NC = 2
NS = 16
L = 16

EC = 100
ET = E // NS
TCH = ET // EC
KB = 20
CPD = E // EC
NP = 10240
SEG = NP // NS
RB = 32

_mesh = plsc.VectorSubcoreMesh(core_axis_name="c", subcore_axis_name="s")


def _agg_body(with_cnt, *refs):
    if with_cnt:
        (srcs, dsts, table, mean_out, inv_out, acc, cnt, sb, db, r0, r1,
         ones, zb, cb, gs0, gs1, ss0, ss1, osem) = refs
    else:
        (srcs, dsts, table, inv_in, mean_out, acc, sb, db, r0, r1,
         zb, cb, gs0, gs1, ss0, ss1) = refs
    d = lax.axis_index("c")
    s = lax.axis_index("s")
    z16 = jnp.zeros((L,), jnp.float32)
    o16 = jnp.ones((L,), jnp.float32)

    def fill_z(r, _):
        for j in range(D // L):
            zb[r, pl.ds(L * j, L)] = z16
        return _
    lax.fori_loop(0, RB, fill_z, None)

    def fill_c(r, _):
        cb[r, pl.ds(0, L)] = z16
        return _
    lax.fori_loop(0, RB, fill_c, None)

    if with_cnt:
        def fill_o(r, _):
            ones[r, pl.ds(0, L)] = o16
            return _
        lax.fori_loop(0, EC, fill_o, None)

    for k in range(SEG // RB):
        pltpu.sync_copy(zb, acc.at[pl.ds(s * SEG + RB * k, RB)])
        if with_cnt:
            pltpu.sync_copy(cb, cnt.at[pl.ds(s * SEG + RB * k, RB)])
    plsc.subcore_barrier()

    rbuf = (r0, r1)
    gsem = (gs0, gs1)
    ssem = (ss0, ss1)

    def block(b, _):
        row = d * CPD + s * TCH + b * KB
        pltpu.sync_copy(srcs.at[pl.ds(row, KB)], sb)
        pltpu.sync_copy(dsts.at[pl.ds(row, KB)], db)
        pg = [None, None]
        psc = [None, None]
        pon = [None, None]
        pg[0] = pltpu.async_copy(table.at[sb.at[0]], r0, gs0)
        for jj in range(KB):
            cur = jj % 2
            nxt = 1 - cur
            if jj + 1 < KB:
                if psc[nxt] is not None:
                    psc[nxt].wait()
                    psc[nxt] = None
                    if with_cnt:
                        pon[nxt].wait()
                pg[nxt] = pltpu.async_copy(
                    table.at[sb.at[jj + 1]], rbuf[nxt], gsem[nxt])
            pg[cur].wait()
            psc[cur] = pltpu.async_copy(
                rbuf[cur], acc.at[db.at[jj]], ssem[cur], add=True)
            if with_cnt:
                pon[cur] = pltpu.async_copy(
                    ones, cnt.at[db.at[jj]], osem, add=True)
        for kk in (0, 1):
            if psc[kk] is not None:
                psc[kk].wait()
                if with_cnt:
                    pon[kk].wait()
        return _
    lax.fori_loop(0, TCH // KB, block, None)
    plsc.subcore_barrier()

    for k in range(SEG // RB):
        rb = s * SEG + RB * k
        pltpu.sync_copy(acc.at[pl.ds(rb, RB)], zb)
        if with_cnt:
            pltpu.sync_copy(cnt.at[pl.ds(rb, RB)], cb)
        else:
            pltpu.sync_copy(inv_in.at[pl.ds(d * NP + rb, RB)], cb)

        def row_mean(r, _):
            cv = cb[r, pl.ds(0, L)]
            if with_cnt:
                inv = 1.0 / jnp.maximum(cv, 1.0)
                cb[r, pl.ds(0, L)] = inv
            else:
                inv = cv
            for j in range(D // L):
                sl = pl.ds(L * j, L)
                zb[r, sl] = zb[r, sl] * inv
            return _
        lax.fori_loop(0, RB, row_mean, None)
        pltpu.sync_copy(zb, mean_out.at[pl.ds(d * NP + rb, RB)])
        if with_cnt:
            pltpu.sync_copy(cb, inv_out.at[pl.ds(d * NP + rb, RB)])


_agg0 = pl.kernel(
    functools.partial(_agg_body, True),
    out_type=[jax.ShapeDtypeStruct((2 * NP, D), jnp.float32),
              jax.ShapeDtypeStruct((2 * NP, L), jnp.float32)],
    mesh=_mesh,
    scratch_types=[
        pltpu.VMEM_SHARED((NP, D), jnp.float32),
        pltpu.VMEM_SHARED((NP, L), jnp.float32),
        pltpu.VMEM((KB, EC), jnp.int32),
        pltpu.VMEM((KB, EC), jnp.int32),
        pltpu.VMEM((EC, D), jnp.float32),
        pltpu.VMEM((EC, D), jnp.float32),
        pltpu.VMEM((EC, L), jnp.float32),
        pltpu.VMEM((RB, D), jnp.float32),
        pltpu.VMEM((RB, L), jnp.float32),
        pltpu.SemaphoreType.DMA,
        pltpu.SemaphoreType.DMA,
        pltpu.SemaphoreType.DMA,
        pltpu.SemaphoreType.DMA,
        pltpu.SemaphoreType.DMA,
    ],
    compiler_params=pltpu.CompilerParams(use_tc_tiling_on_sc=False),
)

_agg1 = pl.kernel(
    functools.partial(_agg_body, False),
    out_type=jax.ShapeDtypeStruct((2 * NP, D), jnp.float32),
    mesh=_mesh,
    scratch_types=[
        pltpu.VMEM_SHARED((NP, D), jnp.float32),
        pltpu.VMEM((KB, EC), jnp.int32),
        pltpu.VMEM((KB, EC), jnp.int32),
        pltpu.VMEM((EC, D), jnp.float32),
        pltpu.VMEM((EC, D), jnp.float32),
        pltpu.VMEM((RB, D), jnp.float32),
        pltpu.VMEM((RB, L), jnp.float32),
        pltpu.SemaphoreType.DMA,
        pltpu.SemaphoreType.DMA,
        pltpu.SemaphoreType.DMA,
        pltpu.SemaphoreType.DMA,
    ],
    compiler_params=pltpu.CompilerParams(use_tc_tiling_on_sc=False),
)


DC = 80
DCH = EL // DC
DW = NC * NS
DCT = DCH // DW
CB = 13
DNB = DCT // CB


def _dec_chunk(sidx_r, didx_r, srows, drows, obuf):
    lane = lax.iota(jnp.int32, L)

    def group(g, _):
        def row16(r16, res):
            r = g * L + r16
            acc = jnp.zeros((L,), jnp.float32)
            for j in range(D // L):
                sl2 = pl.ds(L * j, L)
                acc = acc + srows[r, sl2] * drows[r, sl2]
            dot = jnp.sum(acc)
            return jnp.where(lane == r16, dot, res)
        res = lax.fori_loop(0, L, row16, jnp.zeros((L,), jnp.float32))
        obuf[pl.ds(L * g, L)] = res
        return _
    lax.fori_loop(0, DC // L, group, None)


def _dec_body(z, ell, out, sb, db, s0, s1, d0, d1, obuf,
              sem_s0, sem_s1, sem_d0, sem_d1):
    c = lax.axis_index("c")
    s = lax.axis_index("s")
    w = s * NC + c
    sbuf = (s0, s1)
    dbuf = (d0, d1)
    ssem = (sem_s0, sem_s1)
    dsem = (sem_d0, sem_d1)

    def block(b, _):
        row0 = w * DCT + b * CB
        pltpu.sync_copy(ell.at[pl.ds(row0, CB)], sb)
        pltpu.sync_copy(ell.at[pl.ds(DCH + row0, CB)], db)
        ps = [None, None]
        pd = [None, None]
        ps[0] = pltpu.async_copy(z.at[sb.at[0]], s0, sem_s0)
        pd[0] = pltpu.async_copy(z.at[db.at[0]], d0, sem_d0)
        for jj in range(CB):
            cur = jj % 2
            if jj + 1 < CB:
                nxt = 1 - cur
                ps[nxt] = pltpu.async_copy(
                    z.at[sb.at[jj + 1]], sbuf[nxt], ssem[nxt])
                pd[nxt] = pltpu.async_copy(
                    z.at[db.at[jj + 1]], dbuf[nxt], dsem[nxt])
            ps[cur].wait()
            pd[cur].wait()
            _dec_chunk(None, None, sbuf[cur], dbuf[cur], obuf)
            pltpu.sync_copy(obuf, out.at[pl.ds((row0 + jj) * DC, DC)])
        return _
    lax.fori_loop(0, DNB, block, None)

    crow = jnp.where(w < DCH - DW * DCT, DW * DCT + w, w * DCT + DCT - 1)
    pltpu.sync_copy(ell.at[crow], sb.at[0])
    pltpu.sync_copy(ell.at[DCH + crow], db.at[0])
    pltpu.async_copy(z.at[sb.at[0]], s0, sem_s0).wait()
    pltpu.async_copy(z.at[db.at[0]], d0, sem_d0).wait()
    _dec_chunk(None, None, s0, d0, obuf)
    pltpu.sync_copy(obuf, out.at[pl.ds(crow * DC, DC)])


_dec = pl.kernel(
    _dec_body,
    out_type=jax.ShapeDtypeStruct((EL,), jnp.float32),
    mesh=_mesh,
    scratch_types=[
        pltpu.VMEM((CB, DC), jnp.int32),
        pltpu.VMEM((CB, DC), jnp.int32),
        pltpu.VMEM((DC, D), jnp.float32),
        pltpu.VMEM((DC, D), jnp.float32),
        pltpu.VMEM((DC, D), jnp.float32),
        pltpu.VMEM((DC, D), jnp.float32),
        pltpu.VMEM((DC,), jnp.float32),
        pltpu.SemaphoreType.DMA,
        pltpu.SemaphoreType.DMA,
        pltpu.SemaphoreType.DMA,
        pltpu.SemaphoreType.DMA,
    ],
    compiler_params=pltpu.CompilerParams(
        use_tc_tiling_on_sc=False, needs_layout_passes=False),
)


BN = 1000
NBP = N // BN


def _dense(mean3, x_flat, wm2, wr2, b2, relu):
    def body(m_ref, x_ref, wm_ref, wr_ref, b_ref, o_ref):
        acc = jnp.dot(m_ref[0], wm_ref[0], preferred_element_type=jnp.float32)
        acc = acc + jnp.dot(x_ref[...], wr_ref[0],
                            preferred_element_type=jnp.float32)
        acc = acc + b_ref[0]
        if relu:
            acc = jnp.maximum(acc, 0.0)
        o_ref[...] = acc

    return pl.pallas_call(
        body,
        grid=(2, NBP),
        in_specs=[
            pl.BlockSpec((1, BN, D), lambda h, i: (1 - h, i, 0)),
            pl.BlockSpec((BN, D), lambda h, i: (h * NBP + i, 0)),
            pl.BlockSpec((1, D, D), lambda h, i: (h, 0, 0)),
            pl.BlockSpec((1, D, D), lambda h, i: (h, 0, 0)),
            pl.BlockSpec((1, 1, D), lambda h, i: (h, 0, 0)),
        ],
        out_specs=pl.BlockSpec((BN, D), lambda h, i: (h * NBP + i, 0)),
        out_shape=jax.ShapeDtypeStruct((2 * N, D), jnp.float32),
    )(mean3, x_flat, wm2, wr2, b2)


def kernel(emb_user, emb_item,
           W_msg_ui_0, W_root_ui_0, b_ui_0,
           W_msg_iu_0, W_root_iu_0, b_iu_0,
           W_msg_ui_1, W_root_ui_1, b_ui_1,
           W_msg_iu_1, W_root_iu_1, b_iu_1,
           edge_index_ui, edge_index_iu, edge_label_index):
    table0 = jnp.concatenate([emb_user, emb_item], axis=0)
    srcs = jnp.concatenate([
        edge_index_ui[0], edge_index_iu[0] + N]).astype(jnp.int32)
    srcs = srcs.reshape(2 * CPD, EC)
    dsts = jnp.concatenate([
        edge_index_ui[1], edge_index_iu[1]]).astype(jnp.int32)
    dsts = dsts.reshape(2 * CPD, EC)
    ell2d = jnp.concatenate(
        [edge_label_index[0],
         edge_label_index[1] + N]).astype(jnp.int32).reshape(2 * DCH, DC)

    wm0 = jnp.stack([W_msg_iu_0, W_msg_ui_0])
    wr0 = jnp.stack([W_root_iu_0, W_root_ui_0])
    bb0 = jnp.stack([b_iu_0, b_ui_0]).reshape(2, 1, D)
    wm1 = jnp.stack([W_msg_iu_1, W_msg_ui_1])
    wr1 = jnp.stack([W_root_iu_1, W_root_ui_1])
    bb1 = jnp.stack([b_iu_1, b_ui_1]).reshape(2, 1, D)

    mean0, inv = _agg0(srcs, dsts, table0)
    mean0 = mean0.reshape(2, NP, D)
    table1 = _dense(mean0, table0, wm0, wr0, bb0, True)
    mean1 = _agg1(srcs, dsts, table1, inv).reshape(2, NP, D)
    z_flat = _dense(mean1, table1, wm1, wr1, bb1, False)
    return _dec(z_flat, ell2d)
